# Initial kernel scaffold; baseline (speedup 1.0000x reference)
import jax
import jax.numpy as jnp
from jax.experimental import pallas as pl


def kernel(x, W1, b1, W2, b2):
    raise NotImplementedError("write your pallas kernel here")


# speedup vs baseline: 5.8009x; 5.8009x over previous
import functools

import jax
import jax.numpy as jnp
from jax import lax
from jax.experimental import pallas as pl

D = 16
V = 128
H = 256
IN_DIM = (D - 1) * V
OUT_DIM = D * V


def _mask_w1_kernel(w1_ref, o_ref):
    r = lax.broadcasted_iota(jnp.int32, (IN_DIM, H), 0)
    c = lax.broadcasted_iota(jnp.int32, (IN_DIM, H), 1)
    m = (r // V) <= (c % (D - 1))
    o_ref[...] = jnp.where(m, w1_ref[...], 0.0)


def _mask_w2_kernel(w2_ref, o_ref):
    r = lax.broadcasted_iota(jnp.int32, (H, OUT_DIM), 0)
    c = lax.broadcasted_iota(jnp.int32, (H, OUT_DIM), 1)
    m = (r % (D - 1) + 1) <= (c // V)
    o_ref[...] = jnp.where(m, w2_ref[...], 0.0)


def _made_kernel(x_ref, w1_ref, b1_ref, w2_ref, b2_ref, o_ref, *, bm):
    xb = x_ref[...]
    v_iota = lax.broadcasted_iota(jnp.int32, (bm, V), 1)
    oh_parts = [
        (xb[:, d:d + 1] == v_iota).astype(jnp.float32) for d in range(D - 1)
    ]
    oh_in = jnp.concatenate(oh_parts, axis=1)
    h = jnp.dot(oh_in, w1_ref[...], preferred_element_type=jnp.float32)
    h = jnp.maximum(h + b1_ref[...], 0.0)
    y = jnp.dot(h, w2_ref[...], preferred_element_type=jnp.float32)
    y = y + b2_ref[...]
    acc = jnp.zeros((bm,), jnp.float32)
    for d in range(D):
        e_d = jnp.exp(y[:, d * V:(d + 1) * V])
        oh_d = (xb[:, d:d + 1] == v_iota).astype(jnp.float32)
        norm = jnp.sum(e_d, axis=1)
        sel = jnp.sum(e_d * oh_d, axis=1)
        acc = acc + (jnp.log(sel) - jnp.log(norm))
    o_ref[...] = acc


def kernel(x, W1, b1, W2, b2):
    W1m = pl.pallas_call(
        _mask_w1_kernel,
        out_shape=jax.ShapeDtypeStruct((IN_DIM, H), jnp.float32),
    )(W1)
    W2m = pl.pallas_call(
        _mask_w2_kernel,
        out_shape=jax.ShapeDtypeStruct((H, OUT_DIM), jnp.float32),
    )(W2)
    B = x.shape[0]
    bm = 256
    out = pl.pallas_call(
        functools.partial(_made_kernel, bm=bm),
        grid=(B // bm,),
        in_specs=[
            pl.BlockSpec((bm, D), lambda i: (i, 0)),
            pl.BlockSpec((IN_DIM, H), lambda i: (0, 0)),
            pl.BlockSpec((1, H), lambda i: (0, 0)),
            pl.BlockSpec((H, OUT_DIM), lambda i: (0, 0)),
            pl.BlockSpec((1, OUT_DIM), lambda i: (0, 0)),
        ],
        out_specs=pl.BlockSpec((bm,), lambda i: (i,)),
        out_shape=jax.ShapeDtypeStruct((B,), jnp.float32),
    )(x.astype(jnp.int32), W1m, b1.reshape(1, H), W2m, b2.reshape(1, OUT_DIM))
    return out


# speedup vs baseline: 7.7986x; 1.3444x over previous
import functools

import jax
import jax.numpy as jnp
from jax import lax
from jax.experimental import pallas as pl

D = 16
V = 128
H = 256
IN_DIM = (D - 1) * V
OUT_DIM = D * V


def _mask_w1_kernel(w1_ref, o_ref):
    r = lax.broadcasted_iota(jnp.int32, (IN_DIM, H), 0)
    c = lax.broadcasted_iota(jnp.int32, (IN_DIM, H), 1)
    m = (r // V) <= (c % (D - 1))
    o_ref[...] = jnp.where(m, w1_ref[...], 0.0)


def _mask_w2_kernel(w2_ref, o_ref):
    r = lax.broadcasted_iota(jnp.int32, (H, OUT_DIM), 0)
    c = lax.broadcasted_iota(jnp.int32, (H, OUT_DIM), 1)
    m = (r % (D - 1) + 1) <= (c // V)
    o_ref[...] = jnp.where(m, w2_ref[...], 0.0)


def _made_kernel(x_ref, w1_ref, b1_ref, w2_ref, b2_ref, o_ref, *, bm):
    xb = x_ref[...]
    v_iota = lax.broadcasted_iota(jnp.int32, (bm, V), 1)
    ohs = [(xb[:, d:d + 1] == v_iota).astype(jnp.float32) for d in range(D)]
    oh_in = jnp.concatenate(ohs[:D - 1], axis=1)
    h = jnp.dot(oh_in, w1_ref[...], preferred_element_type=jnp.float32)
    h = jnp.maximum(h + b1_ref[...], 0.0)
    y = jnp.dot(h, w2_ref[...], preferred_element_type=jnp.float32)
    y = y + b2_ref[...]
    ysel = y[:, 0:V] * ohs[0]
    nprod = jnp.sum(jnp.exp(y[:, 0:V]), axis=1)
    for d in range(1, D):
        y_d = y[:, d * V:(d + 1) * V]
        ysel = ysel + y_d * ohs[d]
        nprod = nprod * jnp.sum(jnp.exp(y_d), axis=1)
    o_ref[...] = jnp.sum(ysel, axis=1) - jnp.log(nprod)


def kernel(x, W1, b1, W2, b2):
    W1m = pl.pallas_call(
        _mask_w1_kernel,
        out_shape=jax.ShapeDtypeStruct((IN_DIM, H), jnp.float32),
    )(W1)
    W2m = pl.pallas_call(
        _mask_w2_kernel,
        out_shape=jax.ShapeDtypeStruct((H, OUT_DIM), jnp.float32),
    )(W2)
    B = x.shape[0]
    bm = 256
    out = pl.pallas_call(
        functools.partial(_made_kernel, bm=bm),
        grid=(B // bm,),
        in_specs=[
            pl.BlockSpec((bm, D), lambda i: (i, 0)),
            pl.BlockSpec((IN_DIM, H), lambda i: (0, 0)),
            pl.BlockSpec((1, H), lambda i: (0, 0)),
            pl.BlockSpec((H, OUT_DIM), lambda i: (0, 0)),
            pl.BlockSpec((1, OUT_DIM), lambda i: (0, 0)),
        ],
        out_specs=pl.BlockSpec((bm,), lambda i: (i,)),
        out_shape=jax.ShapeDtypeStruct((B,), jnp.float32),
    )(x.astype(jnp.int32), W1m, b1.reshape(1, H), W2m, b2.reshape(1, OUT_DIM))
    return out


# speedup vs baseline: 8.5573x; 1.0973x over previous
import functools

import jax
import jax.numpy as jnp
from jax import lax
from jax.experimental import pallas as pl

D = 16
V = 128
H = 256
IN_DIM = (D - 1) * V
OUT_DIM = D * V


def _mask_w1_kernel(w1_ref, o_ref):
    r = lax.broadcasted_iota(jnp.int32, (IN_DIM, H), 0)
    c = lax.broadcasted_iota(jnp.int32, (IN_DIM, H), 1)
    m = (r // V) <= (c % (D - 1))
    o_ref[...] = jnp.where(m, w1_ref[...], 0.0)


def _mask_w2_kernel(w2_ref, o_ref):
    r = lax.broadcasted_iota(jnp.int32, (H, OUT_DIM), 0)
    c = lax.broadcasted_iota(jnp.int32, (H, OUT_DIM), 1)
    m = (r % (D - 1) + 1) <= (c // V)
    o_ref[...] = jnp.where(m, w2_ref[...], 0.0)


def _made_kernel(x_ref, w1_ref, b1_ref, w2_ref, b2_ref, o_ref, *, bm):
    xb = x_ref[...]
    v_iota = lax.broadcasted_iota(jnp.int32, (bm, V), 1)
    ohs = [(xb[:, d:d + 1] == v_iota).astype(jnp.float32) for d in range(D)]
    oh_in = jnp.concatenate(ohs[:D - 1], axis=1)
    h = jnp.dot(oh_in, w1_ref[...], preferred_element_type=jnp.float32)
    h = jnp.maximum(h + b1_ref[...], 0.0)
    y = jnp.dot(h, w2_ref[...], preferred_element_type=jnp.float32)
    y = y + b2_ref[...]
    ysel = y[:, 0:V] * ohs[0]
    nprod = jnp.sum(jnp.exp(y[:, 0:V]), axis=1)
    for d in range(1, D):
        y_d = y[:, d * V:(d + 1) * V]
        ysel = ysel + y_d * ohs[d]
        nprod = nprod * jnp.sum(jnp.exp(y_d), axis=1)
    o_ref[...] = jnp.sum(ysel, axis=1) - jnp.log(nprod)


def kernel(x, W1, b1, W2, b2):
    W1m = pl.pallas_call(
        _mask_w1_kernel,
        out_shape=jax.ShapeDtypeStruct((IN_DIM, H), jnp.float32),
    )(W1)
    W2m = pl.pallas_call(
        _mask_w2_kernel,
        out_shape=jax.ShapeDtypeStruct((H, OUT_DIM), jnp.float32),
    )(W2)
    B = x.shape[0]
    bm = 512
    out = pl.pallas_call(
        functools.partial(_made_kernel, bm=bm),
        grid=(B // bm,),
        in_specs=[
            pl.BlockSpec((bm, D), lambda i: (i, 0)),
            pl.BlockSpec((IN_DIM, H), lambda i: (0, 0)),
            pl.BlockSpec((1, H), lambda i: (0, 0)),
            pl.BlockSpec((H, OUT_DIM), lambda i: (0, 0)),
            pl.BlockSpec((1, OUT_DIM), lambda i: (0, 0)),
        ],
        out_specs=pl.BlockSpec((bm,), lambda i: (i,)),
        out_shape=jax.ShapeDtypeStruct((B,), jnp.float32),
    )(x.astype(jnp.int32), W1m, b1.reshape(1, H), W2m, b2.reshape(1, OUT_DIM))
    return out


# speedup vs baseline: 9.5918x; 1.1209x over previous
import functools

import jax
import jax.numpy as jnp
from jax import lax
from jax.experimental import pallas as pl

D = 16
V = 128
H = 256
IN_DIM = (D - 1) * V
OUT_DIM = D * V


def _mask_w1_kernel(w1_ref, o_ref):
    r = lax.broadcasted_iota(jnp.int32, (IN_DIM, H), 0)
    c = lax.broadcasted_iota(jnp.int32, (IN_DIM, H), 1)
    m = (r // V) <= (c % (D - 1))
    o_ref[...] = jnp.where(m, w1_ref[...], 0.0)


def _mask_w2_kernel(w2_ref, o_ref):
    r = lax.broadcasted_iota(jnp.int32, (H, OUT_DIM), 0)
    c = lax.broadcasted_iota(jnp.int32, (H, OUT_DIM), 1)
    m = (r % (D - 1) + 1) <= (c // V)
    o_ref[...] = jnp.where(m, w2_ref[...], 0.0)


def _made_kernel(x_ref, w1_ref, b1_ref, w2_ref, b2_ref, o_ref, *, bm):
    xb = x_ref[...]
    v_iota = lax.broadcasted_iota(jnp.int32, (bm, V), 1)
    ohs = [(xb[:, d:d + 1] == v_iota).astype(jnp.float32) for d in range(D)]
    oh_in = jnp.concatenate(ohs[:D - 1], axis=1)
    h = jnp.dot(oh_in, w1_ref[...], preferred_element_type=jnp.float32)
    h = jnp.maximum(h + b1_ref[...], 0.0)
    y = jnp.dot(h, w2_ref[...], preferred_element_type=jnp.float32)
    y = y + b2_ref[...]
    ysel = y[:, 0:V] * ohs[0]
    nprod = jnp.sum(jnp.exp(y[:, 0:V]), axis=1)
    for d in range(1, D):
        y_d = y[:, d * V:(d + 1) * V]
        ysel = ysel + y_d * ohs[d]
        nprod = nprod * jnp.sum(jnp.exp(y_d), axis=1)
    o_ref[...] = jnp.sum(ysel, axis=1) - jnp.log(nprod)


def kernel(x, W1, b1, W2, b2):
    W1m = pl.pallas_call(
        _mask_w1_kernel,
        out_shape=jax.ShapeDtypeStruct((IN_DIM, H), jnp.float32),
    )(W1)
    W2m = pl.pallas_call(
        _mask_w2_kernel,
        out_shape=jax.ShapeDtypeStruct((H, OUT_DIM), jnp.float32),
    )(W2)
    B = x.shape[0]
    bm = 1024
    out = pl.pallas_call(
        functools.partial(_made_kernel, bm=bm),
        grid=(B // bm,),
        in_specs=[
            pl.BlockSpec((bm, D), lambda i: (i, 0)),
            pl.BlockSpec((IN_DIM, H), lambda i: (0, 0)),
            pl.BlockSpec((1, H), lambda i: (0, 0)),
            pl.BlockSpec((H, OUT_DIM), lambda i: (0, 0)),
            pl.BlockSpec((1, OUT_DIM), lambda i: (0, 0)),
        ],
        out_specs=pl.BlockSpec((bm,), lambda i: (i,)),
        out_shape=jax.ShapeDtypeStruct((B,), jnp.float32),
    )(x.astype(jnp.int32), W1m, b1.reshape(1, H), W2m, b2.reshape(1, OUT_DIM))
    return out
